# baseline (speedup 1.0000x reference)
import jax
import jax.numpy as jnp
from jax import lax
from jax.experimental import pallas as pl
from jax.experimental.pallas import tpu as pltpu
from jax.experimental.pallas import tpu_sc as plsc

NC, NS, L = 2, 16, 16
NW = NC * NS
FEAT_N, D = 16384, 513
CAP = 65536
WIN = FEAT_N // NW
VPW = WIN // L
CHQ = 112
B = 1024
NB = CAP // B
W8 = B + 8
NSL = 4
DH = 256


def _sc_body(feat, hc, nvout, col_v, colf_v, idx_v, vec_v, allcnt_v,
             qbuf, shared, semc, semw):
    cid = lax.axis_index("c")
    sid = lax.axis_index("s")
    wid = cid * NS + sid
    wf = (1 - cid) * NS + sid
    base = wid * WIN
    fbase = wf * WIN

    iota = lax.iota(jnp.int32, L)
    zeros16 = jnp.zeros((L,), jnp.int32)

    pltpu.sync_copy(feat.at[pl.ds(base, WIN), pl.ds(D - 1, 1)], col_v)
    pltpu.sync_copy(feat.at[pl.ds(fbase, WIN), pl.ds(D - 1, 1)], colf_v)

    cnt = jnp.int32(0)
    for v in range(VPW):
        c = plsc.load_gather(col_v, [iota + v * L, zeros16])
        m = c >= 0.0
        mi = m.astype(jnp.int32)
        pos = cnt + plsc.cumsum(mi) - 1
        gidx = iota + (base + v * L)
        plsc.store_scatter(idx_v, [pos], gidx, mask=m)
        cnt = cnt + jnp.sum(mi)

    def _fcount(i, acc):
        m = plsc.load_gather(colf_v, [iota + i * L, zeros16]) >= 0.0
        return acc + jnp.sum(m.astype(jnp.int32))
    fcnt = lax.fori_loop(0, VPW, _fcount, jnp.int32(0))

    vec_v[...] = jnp.full((L,), cnt, jnp.int32)
    pltpu.sync_copy(vec_v, shared.at[pl.ds(wid * L, L)])
    vec_v[...] = jnp.full((L,), fcnt, jnp.int32)
    pltpu.sync_copy(vec_v, shared.at[pl.ds(wf * L, L)])
    plsc.subcore_barrier()
    pltpu.sync_copy(shared, allcnt_v)

    lo = plsc.load_gather(allcnt_v, [iota * L])
    hi = plsc.load_gather(allcnt_v, [(iota + NS) * L])
    dstbase = (jnp.sum(jnp.where(iota < wid, lo, 0))
               + jnp.sum(jnp.where(iota + NS < wid, hi, 0)))
    n_valid = jnp.sum(lo) + jnp.sum(hi)

    @pl.when(wid == 0)
    def _():
        vec_v[...] = jnp.full((L,), n_valid, jnp.int32)
        pltpu.sync_copy(vec_v, nvout)

    NCC = -(-WIN // CHQ)

    for c in range(NCC):
        @pl.when(cnt > CHQ * c)
        def _():
            buf = c % 2
            rsz = jnp.minimum(cnt - CHQ * c, CHQ)

            if c >= 2:
                pltpu.make_async_copy(qbuf.at[0], hc.at[pl.ds(0, CHQ)],
                                      semw).wait()

            def _crow(r, carry):
                g = (r // L) * L
                vec = idx_v[pl.ds(pl.multiple_of(g, 8), L)]
                src = jnp.sum(jnp.where(iota == r - g, vec, 0))
                pltpu.async_copy(feat.at[pl.ds(src, 1)],
                                 qbuf.at[buf].at[pl.ds(r - CHQ * c, 1)],
                                 semc)
                return carry
            lax.fori_loop(CHQ * c, CHQ * c + rsz, _crow, jnp.int32(0))

            def _rdrain(r, carry):
                pltpu.make_async_copy(feat.at[pl.ds(0, 1)],
                                      qbuf.at[0].at[pl.ds(0, 1)],
                                      semc).wait()
                return carry
            lax.fori_loop(0, rsz, _rdrain, jnp.int32(0))

            @pl.when(rsz == CHQ)
            def _():
                pltpu.async_copy(qbuf.at[buf],
                                 hc.at[pl.ds(dstbase + CHQ * c, CHQ)], semw)

            @pl.when(rsz < CHQ)
            def _():
                def _wrow(r, carry):
                    pltpu.async_copy(qbuf.at[buf].at[pl.ds(r, 1)],
                                     hc.at[pl.ds(dstbase + CHQ * c + r, 1)],
                                     semc)
                    return carry
                lax.fori_loop(0, rsz, _wrow, jnp.int32(0))

                def _wdrain(r, carry):
                    pltpu.make_async_copy(feat.at[pl.ds(0, 1)],
                                          qbuf.at[0].at[pl.ds(0, 1)],
                                          semc).wait()
                    return carry
                lax.fori_loop(0, rsz, _wdrain, jnp.int32(0))

    nfull = cnt // CHQ
    nleft = jnp.minimum(nfull, jnp.where(cnt % CHQ > 0, 1, 2))

    def _fdrain(i, carry):
        pltpu.make_async_copy(qbuf.at[0], hc.at[pl.ds(0, CHQ)], semw).wait()
        return carry
    lax.fori_loop(0, nleft, _fdrain, jnp.int32(0))


def _sc_compact(feat):
    run = pl.kernel(
        _sc_body,
        out_type=(jax.ShapeDtypeStruct((FEAT_N, D), jnp.float32),
                  jax.ShapeDtypeStruct((L,), jnp.int32)),
        mesh=plsc.VectorSubcoreMesh(core_axis_name="c", subcore_axis_name="s",
                                    num_cores=NC, num_subcores=NS),
        compiler_params=pltpu.CompilerParams(needs_layout_passes=False,
                                             use_tc_tiling_on_sc=False),
        scratch_types=[
            pltpu.VMEM((WIN, 1), jnp.float32),
            pltpu.VMEM((WIN, 1), jnp.float32),
            pltpu.VMEM((WIN,), jnp.int32),
            pltpu.VMEM((L,), jnp.int32),
            pltpu.VMEM((NW * L,), jnp.int32),
            pltpu.VMEM((2, CHQ, D), jnp.float32),
            pltpu.VMEM_SHARED((NW * L,), jnp.int32),
            pltpu.SemaphoreType.DMA,
            pltpu.SemaphoreType.DMA,
        ],
    )
    return run(feat)


def _floor8(x):
    return (x >> 3) << 3


def _tc_body(nv_ref, hc, queue, out, wbuf, sbuf, obuf, semr, semw):
    i = pl.program_id(0)
    nv = nv_ref[0]

    def params(bi):
        ib = bi * B
        is_tail = ib >= nv
        is_str = jnp.logical_and(ib < nv, nv < ib + B)
        s = jnp.where(is_tail, ib - nv, ib)
        reflen = jnp.where(is_tail, CAP, FEAT_N)
        bs = jnp.minimum(_floor8(s), reflen - W8)
        bs = pl.multiple_of(bs, 8)
        return ib, is_tail, is_str, bs, s - bs

    def fire_reads(bi, slot):
        _, is_tail, is_str, bs, _ = params(bi)

        @pl.when(is_tail)
        def _():
            pltpu.make_async_copy(
                queue.at[pl.ds(bs, W8), pl.ds(0, DH)],
                wbuf.at[slot].at[pl.ds(0, W8), pl.ds(0, DH)],
                semr.at[slot]).start()
            pltpu.make_async_copy(
                queue.at[pl.ds(bs, W8), pl.ds(DH, D - DH)],
                wbuf.at[slot].at[pl.ds(0, W8), pl.ds(DH, D - DH)],
                semr.at[slot]).start()

        @pl.when(jnp.logical_not(is_tail))
        def _():
            pltpu.make_async_copy(
                hc.at[pl.ds(bs, W8), pl.ds(0, DH)],
                wbuf.at[slot].at[pl.ds(0, W8), pl.ds(0, DH)],
                semr.at[slot]).start()
            pltpu.make_async_copy(
                hc.at[pl.ds(bs, W8), pl.ds(DH, D - DH)],
                wbuf.at[slot].at[pl.ds(0, W8), pl.ds(DH, D - DH)],
                semr.at[slot]).start()

        @pl.when(is_str)
        def _():
            pltpu.make_async_copy(queue.at[pl.ds(0, W8)],
                                  sbuf.at[pl.ds(B, W8)],
                                  semr.at[slot]).start()

    def wait_read(slot):
        pltpu.make_async_copy(queue.at[pl.ds(0, W8)], wbuf.at[slot],
                              semr.at[slot]).wait()

    def wait_write(slot):
        pltpu.make_async_copy(obuf.at[slot], out.at[pl.ds(0, B)],
                              semw.at[slot]).wait()

    STRIP = 64

    def roll_store(src_ref, phase, dst_ref):
        def mkbr(k):
            def br():
                for j in range(B // STRIP):
                    dst_ref[pl.ds(j * STRIP, STRIP), :] = (
                        src_ref[j * STRIP + k: j * STRIP + k + STRIP, :])
            return br
        lax.switch(phase, [mkbr(k) for k in range(9)])

    def step(slot):
        @pl.when(i >= NSL)
        def _():
            wait_write(slot)

        @pl.when(i == 0)
        def _():
            fire_reads(0, 0)
            fire_reads(1, 1)

        @pl.when(i + 2 < NB)
        def _():
            fire_reads(i + 2, (i + 2) % NSL)

        ib, is_tail, is_str, bs, phase = params(i)

        wait_read(slot)

        @pl.when(is_str)
        def _():
            wait_read(slot)

        win = wbuf.at[slot]

        @pl.when(jnp.logical_not(is_str))
        def _():
            roll_store(win, phase, obuf.at[slot])

        @pl.when(is_str)
        def _():
            phs = nv - ib
            sp = B - phs
            bp = pl.multiple_of(_floor8(sp), 8)
            tref = sbuf.at[pl.ds(bp, W8)]
            rr = sp - bp
            for j in range(B // STRIP):
                h = lax.switch(phase, [
                    lambda k=k, j=j: win[j * STRIP + k:
                                         j * STRIP + k + STRIP, :]
                    for k in range(9)])
                t = lax.switch(rr, [
                    lambda k=k, j=j: tref[j * STRIP + k:
                                          j * STRIP + k + STRIP, :]
                    for k in range(9)])
                rows = (lax.broadcasted_iota(jnp.int32, (STRIP, D), 0)
                        + j * STRIP)
                obuf[slot, pl.ds(j * STRIP, STRIP), :] = jnp.where(
                    rows < phs, h, t)

        iw = pl.multiple_of(i * B, 8)
        pltpu.make_async_copy(
            obuf.at[slot].at[pl.ds(0, B), pl.ds(0, DH)],
            out.at[pl.ds(iw, B), pl.ds(0, DH)],
            semw.at[slot]).start()
        pltpu.make_async_copy(
            obuf.at[slot].at[pl.ds(0, B), pl.ds(DH, D - DH)],
            out.at[pl.ds(iw, B), pl.ds(DH, D - DH)],
            semw.at[slot]).start()

        @pl.when(i == NB - 1)
        def _():
            for s in range(NSL):
                wait_write((slot - s) % NSL)

    for s in range(NSL):
        @pl.when(i % NSL == s)
        def _(s=s):
            step(s)


def _tc_assemble(nv, hc, queue):
    grid_spec = pltpu.PrefetchScalarGridSpec(
        num_scalar_prefetch=1,
        grid=(NB,),
        in_specs=[pl.BlockSpec(memory_space=pltpu.HBM),
                  pl.BlockSpec(memory_space=pltpu.HBM)],
        out_specs=pl.BlockSpec(memory_space=pltpu.HBM),
        scratch_shapes=[
            pltpu.VMEM((NSL, W8, D), jnp.float32),
            pltpu.VMEM((2 * B + 8, D), jnp.float32),
            pltpu.VMEM((NSL, B, D), jnp.float32),
            pltpu.SemaphoreType.DMA((NSL,)),
            pltpu.SemaphoreType.DMA((NSL,)),
        ],
    )
    return pl.pallas_call(
        _tc_body,
        grid_spec=grid_spec,
        out_shape=jax.ShapeDtypeStruct((CAP, D), jnp.float32),
        compiler_params=pltpu.CompilerParams(
            dimension_semantics=("arbitrary",)),
    )(nv, hc, queue)


def kernel(feat, queue, queue_length):
    del queue_length
    hc, nv = _sc_compact(feat)
    return _tc_assemble(nv, hc, queue)


# speedup vs baseline: 1.0028x; 1.0028x over previous
import jax
import jax.numpy as jnp
from jax import lax
from jax.experimental import pallas as pl
from jax.experimental.pallas import tpu as pltpu
from jax.experimental.pallas import tpu_sc as plsc

NC, NS, L = 2, 16, 16
NW = NC * NS
FEAT_N, D = 16384, 513
CAP = 65536
WIN = FEAT_N // NW
VPW = WIN // L
CHQ = 112
B = 1024
NB = CAP // B
W8 = B + 8
NSL = 4


def _sc_body(feat, hc, nvout, col_v, colf_v, idx_v, vec_v, allcnt_v,
             qbuf, shared, semc, semw):
    cid = lax.axis_index("c")
    sid = lax.axis_index("s")
    wid = cid * NS + sid
    wf = (1 - cid) * NS + sid
    base = wid * WIN
    fbase = wf * WIN

    iota = lax.iota(jnp.int32, L)
    zeros16 = jnp.zeros((L,), jnp.int32)

    pltpu.sync_copy(feat.at[pl.ds(base, WIN), pl.ds(D - 1, 1)], col_v)
    pltpu.sync_copy(feat.at[pl.ds(fbase, WIN), pl.ds(D - 1, 1)], colf_v)

    cnt = jnp.int32(0)
    for v in range(VPW):
        c = plsc.load_gather(col_v, [iota + v * L, zeros16])
        m = c >= 0.0
        mi = m.astype(jnp.int32)
        pos = cnt + plsc.cumsum(mi) - 1
        gidx = iota + (base + v * L)
        plsc.store_scatter(idx_v, [pos], gidx, mask=m)
        cnt = cnt + jnp.sum(mi)

    def _fcount(i, acc):
        m = plsc.load_gather(colf_v, [iota + i * L, zeros16]) >= 0.0
        return acc + jnp.sum(m.astype(jnp.int32))
    fcnt = lax.fori_loop(0, VPW, _fcount, jnp.int32(0))

    vec_v[...] = jnp.full((L,), cnt, jnp.int32)
    pltpu.sync_copy(vec_v, shared.at[pl.ds(wid * L, L)])
    vec_v[...] = jnp.full((L,), fcnt, jnp.int32)
    pltpu.sync_copy(vec_v, shared.at[pl.ds(wf * L, L)])
    plsc.subcore_barrier()
    pltpu.sync_copy(shared, allcnt_v)

    lo = plsc.load_gather(allcnt_v, [iota * L])
    hi = plsc.load_gather(allcnt_v, [(iota + NS) * L])
    dstbase = (jnp.sum(jnp.where(iota < wid, lo, 0))
               + jnp.sum(jnp.where(iota + NS < wid, hi, 0)))
    n_valid = jnp.sum(lo) + jnp.sum(hi)

    @pl.when(wid == 0)
    def _():
        vec_v[...] = jnp.full((L,), n_valid, jnp.int32)
        pltpu.sync_copy(vec_v, nvout)

    NCC = -(-WIN // CHQ)

    for c in range(NCC):
        @pl.when(cnt > CHQ * c)
        def _():
            buf = c % 2
            rsz = jnp.minimum(cnt - CHQ * c, CHQ)

            if c >= 2:
                pltpu.make_async_copy(qbuf.at[0], hc.at[pl.ds(0, CHQ)],
                                      semw).wait()

            def _crow(r, carry):
                g = (r // L) * L
                vec = idx_v[pl.ds(pl.multiple_of(g, 8), L)]
                src = jnp.sum(jnp.where(iota == r - g, vec, 0))
                pltpu.async_copy(feat.at[pl.ds(src, 1)],
                                 qbuf.at[buf].at[pl.ds(r - CHQ * c, 1)],
                                 semc)
                return carry
            lax.fori_loop(CHQ * c, CHQ * c + rsz, _crow, jnp.int32(0))

            def _rdrain(r, carry):
                pltpu.make_async_copy(feat.at[pl.ds(0, 1)],
                                      qbuf.at[0].at[pl.ds(0, 1)],
                                      semc).wait()
                return carry
            lax.fori_loop(0, rsz, _rdrain, jnp.int32(0))

            @pl.when(rsz == CHQ)
            def _():
                pltpu.async_copy(qbuf.at[buf],
                                 hc.at[pl.ds(dstbase + CHQ * c, CHQ)], semw)

            @pl.when(rsz < CHQ)
            def _():
                def _wrow(r, carry):
                    pltpu.async_copy(qbuf.at[buf].at[pl.ds(r, 1)],
                                     hc.at[pl.ds(dstbase + CHQ * c + r, 1)],
                                     semc)
                    return carry
                lax.fori_loop(0, rsz, _wrow, jnp.int32(0))

                def _wdrain(r, carry):
                    pltpu.make_async_copy(feat.at[pl.ds(0, 1)],
                                          qbuf.at[0].at[pl.ds(0, 1)],
                                          semc).wait()
                    return carry
                lax.fori_loop(0, rsz, _wdrain, jnp.int32(0))

    nfull = cnt // CHQ
    nleft = jnp.minimum(nfull, jnp.where(cnt % CHQ > 0, 1, 2))

    def _fdrain(i, carry):
        pltpu.make_async_copy(qbuf.at[0], hc.at[pl.ds(0, CHQ)], semw).wait()
        return carry
    lax.fori_loop(0, nleft, _fdrain, jnp.int32(0))


def _sc_compact(feat):
    run = pl.kernel(
        _sc_body,
        out_type=(jax.ShapeDtypeStruct((FEAT_N, D), jnp.float32),
                  jax.ShapeDtypeStruct((L,), jnp.int32)),
        mesh=plsc.VectorSubcoreMesh(core_axis_name="c", subcore_axis_name="s",
                                    num_cores=NC, num_subcores=NS),
        compiler_params=pltpu.CompilerParams(needs_layout_passes=False,
                                             use_tc_tiling_on_sc=False),
        scratch_types=[
            pltpu.VMEM((WIN, 1), jnp.float32),
            pltpu.VMEM((WIN, 1), jnp.float32),
            pltpu.VMEM((WIN,), jnp.int32),
            pltpu.VMEM((L,), jnp.int32),
            pltpu.VMEM((NW * L,), jnp.int32),
            pltpu.VMEM((2, CHQ, D), jnp.float32),
            pltpu.VMEM_SHARED((NW * L,), jnp.int32),
            pltpu.SemaphoreType.DMA,
            pltpu.SemaphoreType.DMA,
        ],
    )
    return run(feat)


def _floor8(x):
    return (x >> 3) << 3


def _tc_body(nv_ref, hc, queue, out, wbuf, sbuf, obuf, semr, semw):
    i = pl.program_id(0)
    nv = nv_ref[0]

    def params(bi):
        ib = bi * B
        is_tail = ib >= nv
        is_str = jnp.logical_and(ib < nv, nv < ib + B)
        s = jnp.where(is_tail, ib - nv, ib)
        reflen = jnp.where(is_tail, CAP, FEAT_N)
        bs = jnp.minimum(_floor8(s), reflen - W8)
        bs = pl.multiple_of(bs, 8)
        return ib, is_tail, is_str, bs, s - bs

    def fire_reads(bi, slot):
        _, is_tail, is_str, bs, _ = params(bi)

        @pl.when(is_tail)
        def _():
            pltpu.make_async_copy(queue.at[pl.ds(bs, W8)], wbuf.at[slot],
                                  semr.at[slot]).start()

        @pl.when(jnp.logical_not(is_tail))
        def _():
            pltpu.make_async_copy(hc.at[pl.ds(bs, W8)], wbuf.at[slot],
                                  semr.at[slot]).start()

        @pl.when(is_str)
        def _():
            pltpu.make_async_copy(queue.at[pl.ds(0, W8)],
                                  sbuf.at[pl.ds(B, W8)],
                                  semr.at[slot]).start()

    def wait_read(slot):
        pltpu.make_async_copy(queue.at[pl.ds(0, W8)], wbuf.at[slot],
                              semr.at[slot]).wait()

    def wait_write(slot):
        pltpu.make_async_copy(obuf.at[slot], out.at[pl.ds(0, B)],
                              semw.at[slot]).wait()

    STRIP = 64

    def roll_store(src_ref, phase, dst_ref):
        def mkbr(k):
            def br():
                for j in range(B // STRIP):
                    dst_ref[pl.ds(j * STRIP, STRIP), :] = (
                        src_ref[j * STRIP + k: j * STRIP + k + STRIP, :])
            return br
        lax.switch(phase, [mkbr(k) for k in range(9)])

    def step(slot):
        @pl.when(i >= NSL)
        def _():
            wait_write(slot)

        @pl.when(i == 0)
        def _():
            fire_reads(0, 0)
            fire_reads(1, 1)

        @pl.when(i + 2 < NB)
        def _():
            fire_reads(i + 2, (i + 2) % NSL)

        ib, is_tail, is_str, bs, phase = params(i)

        wait_read(slot)

        @pl.when(is_str)
        def _():
            wait_read(slot)

        win = wbuf.at[slot]

        @pl.when(jnp.logical_not(is_str))
        def _():
            roll_store(win, phase, obuf.at[slot])

        @pl.when(is_str)
        def _():
            phs = nv - ib
            sp = B - phs
            bp = pl.multiple_of(_floor8(sp), 8)
            tref = sbuf.at[pl.ds(bp, W8)]
            rr = sp - bp
            for j in range(B // STRIP):
                h = lax.switch(phase, [
                    lambda k=k, j=j: win[j * STRIP + k:
                                         j * STRIP + k + STRIP, :]
                    for k in range(9)])
                t = lax.switch(rr, [
                    lambda k=k, j=j: tref[j * STRIP + k:
                                          j * STRIP + k + STRIP, :]
                    for k in range(9)])
                rows = (lax.broadcasted_iota(jnp.int32, (STRIP, D), 0)
                        + j * STRIP)
                obuf[slot, pl.ds(j * STRIP, STRIP), :] = jnp.where(
                    rows < phs, h, t)

        iw = pl.multiple_of(i * B, 8)
        pltpu.make_async_copy(obuf.at[slot], out.at[pl.ds(iw, B)],
                              semw.at[slot]).start()

        @pl.when(i == NB - 1)
        def _():
            for s in range(NSL):
                wait_write((slot - s) % NSL)

    for s in range(NSL):
        @pl.when(i % NSL == s)
        def _(s=s):
            step(s)


def _tc_assemble(nv, hc, queue):
    grid_spec = pltpu.PrefetchScalarGridSpec(
        num_scalar_prefetch=1,
        grid=(NB,),
        in_specs=[pl.BlockSpec(memory_space=pltpu.HBM),
                  pl.BlockSpec(memory_space=pltpu.HBM)],
        out_specs=pl.BlockSpec(memory_space=pltpu.HBM),
        scratch_shapes=[
            pltpu.VMEM((NSL, W8, D), jnp.float32),
            pltpu.VMEM((2 * B + 8, D), jnp.float32),
            pltpu.VMEM((NSL, B, D), jnp.float32),
            pltpu.SemaphoreType.DMA((NSL,)),
            pltpu.SemaphoreType.DMA((NSL,)),
        ],
    )
    return pl.pallas_call(
        _tc_body,
        grid_spec=grid_spec,
        out_shape=jax.ShapeDtypeStruct((CAP, D), jnp.float32),
        compiler_params=pltpu.CompilerParams(
            dimension_semantics=("arbitrary",)),
    )(nv, hc, queue)


def kernel(feat, queue, queue_length):
    del queue_length
    hc, nv = _sc_compact(feat)
    return _tc_assemble(nv, hc, queue)


# speedup vs baseline: 1.4935x; 1.4894x over previous
import jax
import jax.numpy as jnp
from jax import lax
from jax.experimental import pallas as pl
from jax.experimental.pallas import tpu as pltpu
from jax.experimental.pallas import tpu_sc as plsc

NC, NS, L = 2, 16, 16
NW = NC * NS
FEAT_N, D = 16384, 513
CAP = 65536
WIN = FEAT_N // NW
VPW = WIN // L
CHQ = 112
B = 1024
NB = CAP // B
W8 = B + 8
NSL = 3


def _sc_body(feat, hc, nvout, col_v, colf_v, idx_v, vec_v, allcnt_v,
             qbuf, shared, semc, semw):
    cid = lax.axis_index("c")
    sid = lax.axis_index("s")
    wid = cid * NS + sid
    wf = (1 - cid) * NS + sid
    base = wid * WIN
    fbase = wf * WIN

    iota = lax.iota(jnp.int32, L)
    zeros16 = jnp.zeros((L,), jnp.int32)

    pltpu.sync_copy(feat.at[pl.ds(base, WIN), pl.ds(D - 1, 1)], col_v)
    pltpu.sync_copy(feat.at[pl.ds(fbase, WIN), pl.ds(D - 1, 1)], colf_v)

    cnt = jnp.int32(0)
    for v in range(VPW):
        c = plsc.load_gather(col_v, [iota + v * L, zeros16])
        m = c >= 0.0
        mi = m.astype(jnp.int32)
        pos = cnt + plsc.cumsum(mi) - 1
        gidx = iota + (base + v * L)
        plsc.store_scatter(idx_v, [pos], gidx, mask=m)
        cnt = cnt + jnp.sum(mi)

    def _fcount(i, acc):
        m = plsc.load_gather(colf_v, [iota + i * L, zeros16]) >= 0.0
        return acc + jnp.sum(m.astype(jnp.int32))
    fcnt = lax.fori_loop(0, VPW, _fcount, jnp.int32(0))

    vec_v[...] = jnp.full((L,), cnt, jnp.int32)
    pltpu.sync_copy(vec_v, shared.at[pl.ds(wid * L, L)])
    vec_v[...] = jnp.full((L,), fcnt, jnp.int32)
    pltpu.sync_copy(vec_v, shared.at[pl.ds(wf * L, L)])
    plsc.subcore_barrier()
    pltpu.sync_copy(shared, allcnt_v)

    lo = plsc.load_gather(allcnt_v, [iota * L])
    hi = plsc.load_gather(allcnt_v, [(iota + NS) * L])
    dstbase = (jnp.sum(jnp.where(iota < wid, lo, 0))
               + jnp.sum(jnp.where(iota + NS < wid, hi, 0)))
    n_valid = jnp.sum(lo) + jnp.sum(hi)

    @pl.when(wid == 0)
    def _():
        vec_v[...] = jnp.full((L,), n_valid, jnp.int32)
        pltpu.sync_copy(vec_v, nvout)

    NCC = -(-WIN // CHQ)

    for c in range(NCC):
        @pl.when(cnt > CHQ * c)
        def _():
            buf = c % 2
            rsz = jnp.minimum(cnt - CHQ * c, CHQ)

            if c >= 2:
                pltpu.make_async_copy(qbuf.at[0], hc.at[pl.ds(0, CHQ)],
                                      semw).wait()

            def _crow(r, carry):
                g = (r // L) * L
                vec = idx_v[pl.ds(pl.multiple_of(g, 8), L)]
                src = jnp.sum(jnp.where(iota == r - g, vec, 0))
                pltpu.async_copy(feat.at[pl.ds(src, 1)],
                                 qbuf.at[buf].at[pl.ds(r - CHQ * c, 1)],
                                 semc)
                return carry
            lax.fori_loop(CHQ * c, CHQ * c + rsz, _crow, jnp.int32(0))

            def _rdrain(r, carry):
                pltpu.make_async_copy(feat.at[pl.ds(0, 1)],
                                      qbuf.at[0].at[pl.ds(0, 1)],
                                      semc).wait()
                return carry
            lax.fori_loop(0, rsz, _rdrain, jnp.int32(0))

            @pl.when(rsz == CHQ)
            def _():
                pltpu.async_copy(qbuf.at[buf],
                                 hc.at[pl.ds(dstbase + CHQ * c, CHQ)], semw)

            @pl.when(rsz < CHQ)
            def _():
                def _wrow(r, carry):
                    pltpu.async_copy(qbuf.at[buf].at[pl.ds(r, 1)],
                                     hc.at[pl.ds(dstbase + CHQ * c + r, 1)],
                                     semc)
                    return carry
                lax.fori_loop(0, rsz, _wrow, jnp.int32(0))

                def _wdrain(r, carry):
                    pltpu.make_async_copy(feat.at[pl.ds(0, 1)],
                                          qbuf.at[0].at[pl.ds(0, 1)],
                                          semc).wait()
                    return carry
                lax.fori_loop(0, rsz, _wdrain, jnp.int32(0))

    nfull = cnt // CHQ
    nleft = jnp.minimum(nfull, jnp.where(cnt % CHQ > 0, 1, 2))

    def _fdrain(i, carry):
        pltpu.make_async_copy(qbuf.at[0], hc.at[pl.ds(0, CHQ)], semw).wait()
        return carry
    lax.fori_loop(0, nleft, _fdrain, jnp.int32(0))


def _sc_compact(feat):
    run = pl.kernel(
        _sc_body,
        out_type=(jax.ShapeDtypeStruct((FEAT_N, D), jnp.float32),
                  jax.ShapeDtypeStruct((L,), jnp.int32)),
        mesh=plsc.VectorSubcoreMesh(core_axis_name="c", subcore_axis_name="s",
                                    num_cores=NC, num_subcores=NS),
        compiler_params=pltpu.CompilerParams(needs_layout_passes=False,
                                             use_tc_tiling_on_sc=False),
        scratch_types=[
            pltpu.VMEM((WIN, 1), jnp.float32),
            pltpu.VMEM((WIN, 1), jnp.float32),
            pltpu.VMEM((WIN,), jnp.int32),
            pltpu.VMEM((L,), jnp.int32),
            pltpu.VMEM((NW * L,), jnp.int32),
            pltpu.VMEM((2, CHQ, D), jnp.float32),
            pltpu.VMEM_SHARED((NW * L,), jnp.int32),
            pltpu.SemaphoreType.DMA,
            pltpu.SemaphoreType.DMA,
        ],
    )
    return run(feat)


def _floor8(x):
    return (x >> 3) << 3


def _tc_body(nv_ref, hc, queue, out, wbuf, sbuf, obuf, semr, semw):
    i = pl.program_id(0)
    nv = nv_ref[0]

    def params(bi):
        ib = bi * B
        is_tail = ib >= nv
        is_str = jnp.logical_and(ib < nv, nv < ib + B)
        s = jnp.where(is_tail, ib - nv, ib)
        reflen = jnp.where(is_tail, CAP, FEAT_N)
        bs = jnp.minimum(_floor8(s), reflen - W8)
        bs = pl.multiple_of(bs, 8)
        return ib, is_tail, is_str, bs, s - bs

    def fire_reads(bi, slot):
        _, is_tail, is_str, bs, _ = params(bi)

        @pl.when(is_tail)
        def _():
            pltpu.make_async_copy(queue.at[pl.ds(bs, W8)], wbuf.at[slot],
                                  semr.at[slot]).start()

        @pl.when(jnp.logical_not(is_tail))
        def _():
            pltpu.make_async_copy(hc.at[pl.ds(bs, W8)], wbuf.at[slot],
                                  semr.at[slot]).start()

        @pl.when(is_str)
        def _():
            pltpu.make_async_copy(queue.at[pl.ds(0, W8)],
                                  sbuf.at[pl.ds(B, W8)],
                                  semr.at[slot]).start()

    def wait_read(slot):
        pltpu.make_async_copy(queue.at[pl.ds(0, W8)], wbuf.at[slot],
                              semr.at[slot]).wait()

    def wait_write(slot):
        pltpu.make_async_copy(obuf.at[slot], out.at[pl.ds(0, B)],
                              semw.at[slot]).wait()

    STRIP = 64

    def roll_store(src_ref, phase, dst_ref):
        def mkbr(k):
            def br():
                for j in range(B // STRIP):
                    dst_ref[pl.ds(j * STRIP, STRIP), :] = (
                        src_ref[j * STRIP + k: j * STRIP + k + STRIP, :])
            return br
        lax.switch(phase, [mkbr(k) for k in range(9)])

    def step(slot):
        @pl.when(i >= NSL)
        def _():
            wait_write(slot)

        @pl.when(i == 0)
        def _():
            fire_reads(0, 0)
            fire_reads(1, 1)

        @pl.when(i + 2 < NB)
        def _():
            fire_reads(i + 2, (i + 2) % NSL)

        ib, is_tail, is_str, bs, phase = params(i)

        wait_read(slot)

        @pl.when(is_str)
        def _():
            wait_read(slot)

        win = wbuf.at[slot]

        @pl.when(jnp.logical_not(is_str))
        def _():
            roll_store(win, phase, obuf.at[slot])

        @pl.when(is_str)
        def _():
            phs = nv - ib
            sp = B - phs
            bp = pl.multiple_of(_floor8(sp), 8)
            tref = sbuf.at[pl.ds(bp, W8)]
            rr = sp - bp
            for j in range(B // STRIP):
                h = lax.switch(phase, [
                    lambda k=k, j=j: win[j * STRIP + k:
                                         j * STRIP + k + STRIP, :]
                    for k in range(9)])
                t = lax.switch(rr, [
                    lambda k=k, j=j: tref[j * STRIP + k:
                                          j * STRIP + k + STRIP, :]
                    for k in range(9)])
                rows = (lax.broadcasted_iota(jnp.int32, (STRIP, D), 0)
                        + j * STRIP)
                obuf[slot, pl.ds(j * STRIP, STRIP), :] = jnp.where(
                    rows < phs, h, t)

        iw = pl.multiple_of(i * B, 8)
        pltpu.make_async_copy(obuf.at[slot], out.at[pl.ds(iw, B)],
                              semw.at[slot]).start()

        @pl.when(i == NB - 1)
        def _():
            for s in range(NSL):
                wait_write((slot - s) % NSL)

    for s in range(NSL):
        @pl.when(i % NSL == s)
        def _(s=s):
            step(s)


def _tc_assemble(nv, hc, queue):
    grid_spec = pltpu.PrefetchScalarGridSpec(
        num_scalar_prefetch=1,
        grid=(NB,),
        in_specs=[pl.BlockSpec(memory_space=pltpu.HBM),
                  pl.BlockSpec(memory_space=pltpu.HBM)],
        out_specs=pl.BlockSpec(memory_space=pltpu.HBM),
        scratch_shapes=[
            pltpu.VMEM((NSL, W8, D), jnp.float32),
            pltpu.VMEM((2 * B + 8, D), jnp.float32),
            pltpu.VMEM((NSL, B, D), jnp.float32),
            pltpu.SemaphoreType.DMA((NSL,)),
            pltpu.SemaphoreType.DMA((NSL,)),
        ],
    )
    return pl.pallas_call(
        _tc_body,
        grid_spec=grid_spec,
        out_shape=jax.ShapeDtypeStruct((CAP, D), jnp.float32),
        compiler_params=pltpu.CompilerParams(
            dimension_semantics=("arbitrary",)),
    )(nv, hc, queue)


def kernel(feat, queue, queue_length):
    del queue_length
    hc, nv = _sc_compact(feat)
    return _tc_assemble(nv, hc, queue)
